# baseline (speedup 1.0000x reference)
import jax
import jax.numpy as jnp
from jax.experimental import pallas as pl

N_PAPER = 10000
N_AUTHOR = 10000
E = 160000
F_IN = 128
HID = 128
HEADS = 8
DH = HID // HEADS
OUT = 16


def _edge_softmax(alpha, dst, num_nodes):
    amax = jax.ops.segment_max(alpha, dst, num_segments=num_nodes)
    amax = jnp.where(jnp.isfinite(amax), amax, 0.0)
    e = jnp.exp(alpha - amax[dst])
    denom = jax.ops.segment_sum(e, dst, num_segments=num_nodes)
    return e / (denom[dst] + 1e-16)


def _han_propagate(x_src_h, x_dst_h, att_src, att_dst, edge_index, num_dst):
    src, dst = edge_index[0], edge_index[1]
    a_src = (x_src_h * att_src[None, :, :]).sum(-1)
    a_dst = (x_dst_h * att_dst[None, :, :]).sum(-1)
    alpha = a_src[src] + a_dst[dst]
    alpha = jax.nn.leaky_relu(alpha, 0.2)
    alpha = _edge_softmax(alpha, dst, num_dst)
    msg = x_src_h[src] * alpha[:, :, None]
    out = jax.ops.segment_sum(msg, dst, num_segments=num_dst)
    return jax.nn.relu(out)


def _semantic_group(outs, q, k_w, k_b):
    out = jnp.stack([o.reshape(o.shape[0], -1) for o in outs], axis=0)
    k = jnp.tanh(out @ k_w + k_b[None, None, :]).mean(axis=1)
    score = (q[None, :] * k).sum(-1)
    attn = jax.nn.softmax(score, axis=0)
    return (out * attn[:, None, None]).sum(0)


def _out_mm_body(feat_ref, w_ref, b_ref, o_ref):
    x = feat_ref[...]
    x = jnp.where(x > 0, x, jnp.exp(x) - 1.0)
    o_ref[...] = jnp.dot(x, w_ref[...], preferred_element_type=jnp.float32) + b_ref[...]


def _out_matmul(feat, w, b):
    blk = 1000
    grid = feat.shape[0] // blk
    return pl.pallas_call(
        _out_mm_body,
        grid=(grid,),
        in_specs=[
            pl.BlockSpec((blk, HID), lambda i: (i, 0)),
            pl.BlockSpec((HID, OUT), lambda i: (0, 0)),
            pl.BlockSpec((1, OUT), lambda i: (0, 0)),
        ],
        out_specs=pl.BlockSpec((blk, OUT), lambda i: (i, 0)),
        out_shape=jax.ShapeDtypeStruct((feat.shape[0], OUT), jnp.float32),
    )(feat, w, b.reshape(1, OUT))


def kernel(x_paper, x_author, edge_index_writes, edge_index_cites, W_paper, b_paper, W_author, b_author, att_src_writes, att_dst_writes, att_src_cites, att_dst_cites, k_w, k_b, q_sem, out_w, out_b):
    h_paper = (x_paper @ W_paper + b_paper[None, :]).reshape(-1, HEADS, DH)
    h_author = (x_author @ W_author + b_author[None, :]).reshape(-1, HEADS, DH)
    out_writes = _han_propagate(h_author, h_paper, att_src_writes, att_dst_writes, edge_index_writes, N_PAPER)
    out_cites = _han_propagate(h_paper, h_paper, att_src_cites, att_dst_cites, edge_index_cites, N_PAPER)
    paper_out = _semantic_group([out_writes, out_cites], q_sem, k_w, k_b)
    return _out_matmul(paper_out, out_w, out_b)


# speedup vs baseline: 22.8255x; 22.8255x over previous
import functools

import jax
import jax.numpy as jnp
from jax import lax
from jax.experimental import pallas as pl
from jax.experimental.pallas import tpu as pltpu
from jax.experimental.pallas import tpu_sc as plsc

N_PAPER = 10000
N_AUTHOR = 10000
E = 160000
F_IN = 128
HID = 128
HEADS = 8
DH = HID // HEADS
OUT = 16

NPAD = 10240
EPAD = 163840
N_TILES = 32
EPT = EPAD // N_TILES
CHUNK = 32
NCHUNKS = EPT // CHUNK
ROWS_PT = NPAD // 16
RBLK = 2560
ACCW = 144


def _proj_body(xp_ref, xa_ref, wp_ref, bp_ref, wa_ref, ba_ref,
               mws_ref, mwd_ref, mcs_ref, mcd_ref,
               hp_ref, ha_ref, aw_ref, bw_ref, ac_ref, bc_ref):
    hp = jnp.dot(xp_ref[...], wp_ref[...], preferred_element_type=jnp.float32) + bp_ref[...]
    ha = jnp.dot(xa_ref[...], wa_ref[...], preferred_element_type=jnp.float32) + ba_ref[...]
    hp_ref[...] = hp
    ha_ref[...] = ha
    hi = jax.lax.Precision.HIGHEST
    aw_ref[...] = jnp.dot(ha, mws_ref[...], preferred_element_type=jnp.float32, precision=hi)
    bw_ref[...] = jnp.dot(hp, mwd_ref[...], preferred_element_type=jnp.float32, precision=hi)
    ac_ref[...] = jnp.dot(hp, mcs_ref[...], preferred_element_type=jnp.float32, precision=hi)
    bc_ref[...] = jnp.dot(hp, mcd_ref[...], preferred_element_type=jnp.float32, precision=hi)


def _projections(xp, xa, wp, bp, wa, ba, mws, mwd, mcs, mcd):
    grid = NPAD // RBLK
    blk = lambda r, c: pl.BlockSpec((r, c), lambda i: (i, 0))
    full = lambda r, c: pl.BlockSpec((r, c), lambda i: (0, 0))
    return pl.pallas_call(
        _proj_body,
        grid=(grid,),
        in_specs=[blk(RBLK, 128), blk(RBLK, 128),
                  full(128, 128), full(1, 128), full(128, 128), full(1, 128),
                  full(128, 128), full(128, 128), full(128, 128), full(128, 128)],
        out_specs=[blk(RBLK, 128), blk(RBLK, 128),
                   blk(RBLK, 128), blk(RBLK, 128), blk(RBLK, 128), blk(RBLK, 128)],
        out_shape=[jax.ShapeDtypeStruct((NPAD, 128), jnp.float32),
                   jax.ShapeDtypeStruct((NPAD, 128), jnp.float32),
                   jax.ShapeDtypeStruct((NPAD, 128), jnp.float32),
                   jax.ShapeDtypeStruct((NPAD, 128), jnp.float32),
                   jax.ShapeDtypeStruct((NPAD, 128), jnp.float32),
                   jax.ShapeDtypeStruct((NPAD, 128), jnp.float32)],
    )(xp, xa, wp, bp.reshape(1, 128), wa, ba.reshape(1, 128), mws, mwd, mcs, mcd)


def _sc_edges_body(src_w, dst_w, src_c, dst_c,
                   aw, bw, ac, bc, hw, hc,
                   acc_out, den_out, e_out,
                   srci, dsti, zidx, arows, brows, hrows, accb, accb2,
                   ebuf, ebuf2, zbuf,
                   acc, sema, semb, semh):
    c = lax.axis_index("c")
    s = lax.axis_index("s")
    tile_base = (c * 16 + s) * EPT
    row0 = s * ROWS_PT
    out_base = c * NPAD + row0

    def _fill_zidx(base):
        for g in range(CHUNK // 16):
            zidx[pl.ds(g * 16, 16)] = lax.iota(jnp.int32, 16) + (base + g * 16)

    def _zero_rows(buf, n):
        def _z(i, _):
            for g in range(128 // 16):
                buf[i, pl.ds(g * 16, 16)] = jnp.zeros((16,), jnp.float32)
            return 0
        lax.fori_loop(0, n, _z, 0)

    def _zero_acc():
        for k in range(ROWS_PT // CHUNK):
            _fill_zidx(row0 + k * CHUNK)
            pltpu.sync_copy(zbuf, acc.at[zidx])

    _zero_rows(zbuf, CHUNK)

    for t in range(2):
        e_src = src_w if t == 0 else src_c
        e_dst = dst_w if t == 0 else dst_c
        a_tab = aw if t == 0 else ac
        b_tab = bw if t == 0 else bc
        h_tab = hw if t == 0 else hc

        _zero_acc()
        plsc.subcore_barrier()

        def _do_chunk(j, ab, eb):
            off = tile_base + j * CHUNK
            pltpu.sync_copy(e_src.at[pl.ds(off, CHUNK)], srci)
            pltpu.sync_copy(e_dst.at[pl.ds(off, CHUNK)], dsti)
            ca = pltpu.async_copy(a_tab.at[srci], arows, sema)
            cb = pltpu.async_copy(b_tab.at[dsti], brows, semb)
            ch = pltpu.async_copy(h_tab.at[srci], hrows, semh)
            ca.wait()
            cb.wait()
            ch.wait()

            def _edge(i, _):
                al = arows[i, pl.ds(0, 16)] + brows[i, pl.ds(0, 16)]
                al = jnp.where(al >= 0.0, al, 0.2 * al)
                ev = jnp.exp(al)
                eb[i, :] = ev
                for h in range(HEADS):
                    hv = hrows[i, pl.ds(h * 16, 16)]
                    ab[i, pl.ds(h * 16, 16)] = hv * ev[h]
                return 0
            lax.fori_loop(0, CHUNK, _edge, 0)
            pltpu.sync_copy(ab, acc.at[dsti], add=True)
            pltpu.sync_copy(eb, e_out.at[t, pl.ds(off, CHUNK), :])

        def _chunk2(jj, _):
            _do_chunk(2 * jj, accb2, ebuf2)
            _do_chunk(2 * jj + 1, accb, ebuf)
            return 0

        lax.fori_loop(0, NCHUNKS // 2, _chunk2, 0)
        plsc.subcore_barrier()

        for k in range(ROWS_PT // CHUNK):
            _fill_zidx(row0 + k * CHUNK)
            pltpu.async_copy(acc.at[zidx], accb, sema).wait()
            pltpu.sync_copy(accb,
                            acc_out.at[t, pl.ds(out_base + k * CHUNK, CHUNK), :])
        plsc.subcore_barrier()

    _zero_acc()
    pltpu.async_copy(acc.at[zidx], hrows, sema).wait()
    _zero_rows(accb, CHUNK)
    _zero_rows(accb2, CHUNK)
    plsc.subcore_barrier()

    for t in range(2):
        e_dst = dst_w if t == 0 else dst_c

        def _den_chunk(j, ab):
            off = tile_base + j * CHUNK
            pltpu.sync_copy(e_dst.at[pl.ds(off, CHUNK)], dsti)
            pltpu.sync_copy(e_out.at[t, pl.ds(off, CHUNK), :], ebuf)

            def _edge_d(i, _):
                ab[i, pl.ds(t * 16, 16)] = ebuf[i, :]
                return 0
            lax.fori_loop(0, CHUNK, _edge_d, 0)
            pltpu.sync_copy(ab, acc.at[dsti], add=True)

        def _den_chunk2(jj, _):
            _den_chunk(2 * jj, accb2)
            _den_chunk(2 * jj + 1, accb)
            return 0

        lax.fori_loop(0, NCHUNKS // 2, _den_chunk2, 0)
        plsc.subcore_barrier()
        pltpu.async_copy(acc.at[zidx], hrows, sema).wait()
        _zero_rows(accb, CHUNK)
        _zero_rows(accb2, CHUNK)
        plsc.subcore_barrier()

    for k in range(ROWS_PT // CHUNK):
        _fill_zidx(row0 + k * CHUNK)
        pltpu.async_copy(acc.at[zidx], accb, sema).wait()
        pltpu.sync_copy(accb, den_out.at[pl.ds(out_base + k * CHUNK, CHUNK), :])
    plsc.subcore_barrier()


def _sc_edges(src_w, dst_w, src_c, dst_c, aw, bw, ac, bc, hw, hc):
    mesh = plsc.VectorSubcoreMesh(core_axis_name="c", subcore_axis_name="s")
    fn = functools.partial(
        pl.kernel,
        mesh=mesh,
        out_type=[jax.ShapeDtypeStruct((2, 2 * NPAD, 128), jnp.float32),
                  jax.ShapeDtypeStruct((2 * NPAD, 128), jnp.float32),
                  jax.ShapeDtypeStruct((2, EPAD, 16), jnp.float32)],
        scratch_types=[
            pltpu.VMEM((CHUNK,), jnp.int32),
            pltpu.VMEM((CHUNK,), jnp.int32),
            pltpu.VMEM((CHUNK,), jnp.int32),
            pltpu.VMEM((CHUNK, 128), jnp.float32),
            pltpu.VMEM((CHUNK, 128), jnp.float32),
            pltpu.VMEM((CHUNK, 128), jnp.float32),
            pltpu.VMEM((CHUNK, 128), jnp.float32),
            pltpu.VMEM((CHUNK, 128), jnp.float32),
            pltpu.VMEM((CHUNK, 16), jnp.float32),
            pltpu.VMEM((CHUNK, 16), jnp.float32),
            pltpu.VMEM((CHUNK, 128), jnp.float32),
            pltpu.VMEM_SHARED((NPAD, 128), jnp.float32),
            pltpu.SemaphoreType.DMA,
            pltpu.SemaphoreType.DMA,
            pltpu.SemaphoreType.DMA,
        ],
    )(_sc_edges_body)
    return fn(src_w, dst_w, src_c, dst_c, aw, bw, ac, bc, hw, hc)


def _merge_body(acc0_ref, acc1_ref, den0_ref, den1_ref, emw_ref, emc_ref,
                kw_ref, kb_ref, feats_ref, ksums_ref):
    i = pl.program_id(0)
    rows = lax.broadcasted_iota(jnp.int32, (RBLK, 1), 0) + i * RBLK
    den = den0_ref[...] + den1_ref[...]
    parts = []
    for t in range(2):
        num = acc0_ref[t] + acc1_ref[t]
        em = emw_ref if t == 0 else emc_ref
        dexp = jnp.dot(den, em[...], preferred_element_type=jnp.float32)
        out = jnp.maximum(num / (dexp + 1e-16), 0.0)
        feats_ref[t] = out
        kk = jnp.tanh(jnp.dot(out, kw_ref[...], preferred_element_type=jnp.float32)
                      + kb_ref[...])
        kk = jnp.where(rows < N_PAPER, kk, 0.0)
        parts.append(jnp.sum(kk, axis=0, keepdims=True))
    part = jnp.concatenate(parts, axis=0)

    @pl.when(i == 0)
    def _():
        ksums_ref[...] = part

    @pl.when(i != 0)
    def _():
        ksums_ref[...] = ksums_ref[...] + part


def _merge(acc0, acc1, den0, den1, emw, emc, kw, kb):
    grid = (NPAD // RBLK,)
    return pl.pallas_call(
        _merge_body,
        grid=grid,
        in_specs=[
            pl.BlockSpec((2, RBLK, 128), lambda i: (0, i, 0)),
            pl.BlockSpec((2, RBLK, 128), lambda i: (0, i, 0)),
            pl.BlockSpec((RBLK, 128), lambda i: (i, 0)),
            pl.BlockSpec((RBLK, 128), lambda i: (i, 0)),
            pl.BlockSpec((128, 128), lambda i: (0, 0)),
            pl.BlockSpec((128, 128), lambda i: (0, 0)),
            pl.BlockSpec((128, 128), lambda i: (0, 0)),
            pl.BlockSpec((1, 128), lambda i: (0, 0)),
        ],
        out_specs=[
            pl.BlockSpec((2, RBLK, 128), lambda i: (0, i, 0)),
            pl.BlockSpec((2, 128), lambda i: (0, 0)),
        ],
        out_shape=[jax.ShapeDtypeStruct((2, NPAD, 128), jnp.float32),
                   jax.ShapeDtypeStruct((2, 128), jnp.float32)],
    )(acc0, acc1, den0, den1, emw, emc, kw, kb.reshape(1, 128))


def _out_body(feats_ref, attn_ref, ow_ref, ob_ref, o_ref):
    a0 = attn_ref[0]
    a1 = attn_ref[1]
    comb = feats_ref[0] * a0 + feats_ref[1] * a1
    comb = jnp.where(comb > 0.0, comb, jnp.exp(comb) - 1.0)
    o_ref[...] = jnp.dot(comb, ow_ref[...], preferred_element_type=jnp.float32) + ob_ref[...]


def _out_stage(feats, attn, ow, ob):
    grid = (NPAD // RBLK,)
    return pl.pallas_call(
        _out_body,
        grid=grid,
        in_specs=[
            pl.BlockSpec((2, RBLK, 128), lambda i: (0, i, 0)),
            pl.BlockSpec(memory_space=pltpu.SMEM),
            pl.BlockSpec((128, OUT), lambda i: (0, 0)),
            pl.BlockSpec((1, OUT), lambda i: (0, 0)),
        ],
        out_specs=pl.BlockSpec((RBLK, OUT), lambda i: (i, 0)),
        out_shape=jax.ShapeDtypeStruct((NPAD, OUT), jnp.float32),
    )(feats, attn, ow, ob.reshape(1, OUT))


def _att_embed(att):
    r = jnp.arange(HID)
    return jnp.zeros((HID, 128), jnp.float32).at[r, r // DH].set(att.reshape(-1))


def kernel(x_paper, x_author, edge_index_writes, edge_index_cites, W_paper, b_paper, W_author, b_author, att_src_writes, att_dst_writes, att_src_cites, att_dst_cites, k_w, k_b, q_sem, out_w, out_b):
    f32 = jnp.float32
    xp = jnp.pad(x_paper, ((0, NPAD - N_PAPER), (0, 0)))
    xa = jnp.pad(x_author, ((0, NPAD - N_AUTHOR), (0, 0)))

    mws = _att_embed(att_src_writes)
    mwd = _att_embed(att_dst_writes)
    mcs = _att_embed(att_src_cites)
    mcd = _att_embed(att_dst_cites)

    h_paper, h_author, aw, bw, ac, bc = _projections(
        xp, xa, W_paper, b_paper, W_author, b_author, mws, mwd, mcs, mcd)

    npad_e = EPAD - E
    pad_src = jnp.zeros((npad_e,), jnp.int32)
    pad_dst = jnp.full((npad_e,), N_PAPER, jnp.int32)
    src_w = jnp.concatenate([edge_index_writes[0], pad_src])
    dst_w = jnp.concatenate([edge_index_writes[1], pad_dst])
    src_c = jnp.concatenate([edge_index_cites[0], pad_src])
    dst_c = jnp.concatenate([edge_index_cites[1], pad_dst])

    acc, den, _e_spill = _sc_edges(src_w, dst_w, src_c, dst_c, aw, bw, ac, bc,
                                   h_author, h_paper)

    col = jnp.arange(HID)
    emw = jnp.zeros((HID, HID), f32).at[col // DH, col].set(1.0)
    emc = jnp.zeros((HID, HID), f32).at[16 + col // DH, col].set(1.0)

    feats, ksums = _merge(acc[:, :NPAD], acc[:, NPAD:], den[:NPAD], den[NPAD:],
                          emw, emc, k_w, k_b)

    k_mean = ksums / float(N_PAPER)
    score = (q_sem[None, :] * k_mean).sum(-1)
    attn = jax.nn.softmax(score, axis=0)

    out = _out_stage(feats, attn, out_w, out_b)
    return out[:N_PAPER]
